# Initial kernel scaffold; baseline (speedup 1.0000x reference)
import jax
import jax.numpy as jnp
from jax.experimental import pallas as pl


def kernel(points_2d, camera_indices, point_indices, camera_params, points_3d):
    raise NotImplementedError("write your pallas kernel here")


# speedup vs baseline: 2.9689x; 2.9689x over previous
import functools

import jax
import jax.numpy as jnp
from jax import lax
from jax.experimental import pallas as pl
from jax.experimental.pallas import tpu as pltpu
from jax.experimental.pallas import tpu_sc as plsc

NC, NS, L = 2, 16, 16
NW = NC * NS


def _ceil_to(x, m):
    return (x + m - 1) // m * m


@functools.lru_cache(maxsize=None)
def _make_kernel(n_obs, n_cam, n_pts):
    B = 3136
    nsteps = -(-n_obs // (NW * B))
    C = nsteps * B
    S = _ceil_to(-(-(n_obs - C) // (NW - 1)), 16)
    last = n_obs - C

    mesh = plsc.VectorSubcoreMesh(
        core_axis_name="c", subcore_axis_name="s",
        num_cores=NC, num_subcores=NS)

    @functools.partial(
        pl.kernel,
        out_type=jax.ShapeDtypeStruct((2 * n_obs,), jnp.float32),
        mesh=mesh,
        compiler_params=pltpu.CompilerParams(needs_layout_passes=False),
        scratch_types=[
            pltpu.VMEM((10 * n_cam,), jnp.float32),
            pltpu.VMEM((B,), jnp.int32),
            pltpu.VMEM((B,), jnp.int32),
            pltpu.VMEM((B,), jnp.float32),
            pltpu.VMEM((B,), jnp.float32),
            pltpu.VMEM((B,), jnp.float32),
            pltpu.VMEM((2 * B,), jnp.float32),
            pltpu.VMEM((2 * B,), jnp.float32),
            pltpu.SemaphoreType.DMA,
        ],
    )
    def reproj(p2d_hbm, cidx_hbm, pidx_hbm, cam_hbm, xs_hbm, ys_hbm, zs_hbm,
               out_hbm,
               cam_v, cidx_v, pidx_v, px_v, py_v, pz_v, obs_v, out_v, sem):
        wid = lax.axis_index("s") * NC + lax.axis_index("c")
        base = jnp.minimum(wid * S, last)
        pltpu.sync_copy(cam_hbm, cam_v)
        iota = lax.iota(jnp.int32, L)

        def step(si, carry):
            off = pl.multiple_of(base + si * B, 16)
            pltpu.sync_copy(pidx_hbm.at[pl.ds(off, B)], pidx_v)
            cx = pltpu.async_copy(xs_hbm.at[pidx_v], px_v, sem)
            cy = pltpu.async_copy(ys_hbm.at[pidx_v], py_v, sem)
            cz = pltpu.async_copy(zs_hbm.at[pidx_v], pz_v, sem)
            pltpu.sync_copy(cidx_hbm.at[pl.ds(off, B)], cidx_v)
            pltpu.sync_copy(p2d_hbm.at[pl.ds(2 * off, 2 * B)], obs_v)
            cx.wait()
            cy.wait()
            cz.wait()

            def inner(k, c):
                rows = k * L + iota
                rows2 = rows + rows
                ci = cidx_v[pl.ds(k * L, L)]
                cb = ci * 10
                qw = plsc.load_gather(cam_v, [cb])
                qx = plsc.load_gather(cam_v, [cb + 1])
                qy = plsc.load_gather(cam_v, [cb + 2])
                qz = plsc.load_gather(cam_v, [cb + 3])
                tx = plsc.load_gather(cam_v, [cb + 4])
                ty = plsc.load_gather(cam_v, [cb + 5])
                tz = plsc.load_gather(cam_v, [cb + 6])
                fo = plsc.load_gather(cam_v, [cb + 7])
                k1 = plsc.load_gather(cam_v, [cb + 8])
                k2 = plsc.load_gather(cam_v, [cb + 9])
                px = px_v[pl.ds(k * L, L)]
                py = py_v[pl.ds(k * L, L)]
                pz = pz_v[pl.ds(k * L, L)]
                s = qw * qw + qx * qx + qy * qy + qz * qz
                inv = 2.0 / s
                t1 = qy * pz - qz * py + qw * px
                t2 = qz * px - qx * pz + qw * py
                t3 = qx * py - qy * px + qw * pz
                c1 = qy * t3 - qz * t2
                c2 = qz * t1 - qx * t3
                c3 = qx * t2 - qy * t1
                x = px + inv * c1 + tx
                y = py + inv * c2 + ty
                z = pz + inv * c3 + tz
                invz = -1.0 / z
                u = x * invz
                v = y * invz
                n = u * u + v * v
                r = 1.0 + k1 * n + k2 * (n * n)
                rf = r * fo
                ox = plsc.load_gather(obs_v, [rows2])
                oy = plsc.load_gather(obs_v, [rows2 + 1])
                plsc.store_scatter(out_v, [rows2], u * rf - ox)
                plsc.store_scatter(out_v, [rows2 + 1], v * rf - oy)
                return c

            lax.fori_loop(0, B // L, inner, 0)
            pltpu.sync_copy(out_v, out_hbm.at[pl.ds(2 * off, 2 * B)])
            return carry

        lax.fori_loop(0, nsteps, step, 0)

    return reproj


def kernel(points_2d, camera_indices, point_indices, camera_params, points_3d):
    n_obs = points_2d.shape[0]
    fn = _make_kernel(n_obs, camera_params.shape[0], points_3d.shape[0])
    pts = points_3d.astype(jnp.float32)
    out_flat = fn(points_2d.astype(jnp.float32).reshape(-1),
                  camera_indices.astype(jnp.int32),
                  point_indices.astype(jnp.int32),
                  camera_params.astype(jnp.float32).reshape(-1),
                  pts[:, 0] + 0.0,
                  pts[:, 1] + 0.0,
                  pts[:, 2] + 0.0)
    return out_flat.reshape(n_obs, 2)
